# Initial kernel scaffold; baseline (speedup 1.0000x reference)
import jax
import jax.numpy as jnp
from jax.experimental import pallas as pl


def kernel(node_embed, adj_matrix, weight_p):
    raise NotImplementedError("write your pallas kernel here")


# speedup vs baseline: 1.0176x; 1.0176x over previous
import functools

import jax
import jax.numpy as jnp
from jax import lax
from jax.experimental import pallas as pl
from jax.experimental.pallas import tpu as pltpu
from jax.experimental.pallas import tpu_sc as plsc

B, N, F = 2, 4096, 128
K = 2048
RT = 4
RTS = K // RT
JT = 8
JTS = N // JT

NC = 2
NS = 16
NW = NC * NS
ROWS_PER_W = (B * K) // NW
RPB = K // NW
CHUNK = 8


def _topk_tc_body(x_ref, w_ref, pooled_ref, idx_ref, srow_s, scol_s, rank_s):
    rt = pl.program_id(1)
    x = x_ref[0]

    @pl.when(rt == 0)
    def _():
        w = w_ref[...]
        scol = jnp.dot(x, w)
        srow = lax.dot_general(w, x, (((0,), (1,)), ((), ())))
        scol_s[...] = scol
        srow_s[...] = srow
        ii = lax.broadcasted_iota(jnp.int32, (N, 1), 0)
        ones = jnp.ones((JTS, 1), jnp.float32)

        def rank_step(jt, acc):
            sj = srow_s[:, pl.ds(jt * JTS, JTS)]
            jj = jt * JTS + lax.broadcasted_iota(jnp.int32, (1, JTS), 1)
            gt = sj > scol
            tie = (sj == scol) & (jj < ii)
            contrib = jnp.where(gt | tie, 1.0, 0.0)
            return acc + jnp.dot(contrib, ones,
                                 precision=lax.Precision.HIGHEST)

        rank_s[...] = lax.fori_loop(
            0, JT, rank_step, jnp.zeros((N, 1), jnp.float32))

    rank = rank_s[...]
    rr = (rt * RTS + lax.broadcasted_iota(jnp.int32, (1, RTS), 1)
          ).astype(jnp.float32)
    sel = jnp.where(rank == rr, 1.0, 0.0)
    iota_row = lax.broadcasted_iota(jnp.int32, (1, N), 1).astype(jnp.float32)
    idx_f = jnp.dot(iota_row, sel, precision=lax.Precision.HIGHEST)
    idx_ref[0] = idx_f.astype(jnp.int32)
    vals = lax.dot_general(sel, scol_s[...], (((0,), (0,)), ((), ())),
                           precision=lax.Precision.HIGHEST)
    pooled = lax.dot_general(sel, x, (((0,), (0,)), ((), ())),
                             precision=lax.Precision.HIGHEST)
    pooled_ref[0] = pooled * jnp.tanh(vals)


def _topk_pool_tc(node_embed, weight_p, interpret=False):
    return pl.pallas_call(
        _topk_tc_body,
        grid=(B, RT),
        in_specs=[
            pl.BlockSpec((1, N, F), lambda b, rt: (b, 0, 0)),
            pl.BlockSpec((F, 1), lambda b, rt: (0, 0)),
        ],
        out_specs=[
            pl.BlockSpec((1, RTS, F), lambda b, rt: (b, rt, 0)),
            pl.BlockSpec((1, 1, RTS), lambda b, rt: (b, 0, rt)),
        ],
        out_shape=[
            jax.ShapeDtypeStruct((B, K, F), jnp.float32),
            jax.ShapeDtypeStruct((B, 1, K), jnp.int32),
        ],
        scratch_shapes=[
            pltpu.VMEM((1, N), jnp.float32),
            pltpu.VMEM((N, 1), jnp.float32),
            pltpu.VMEM((N, 1), jnp.float32),
        ],
        compiler_params=None if interpret else pltpu.CompilerParams(
            vmem_limit_bytes=60 * 1024 * 1024),
        interpret=interpret,
    )(node_embed, weight_p)


def _adj_sc_body(adj_hbm, idx_hbm, out_hbm,
                 colidx_v, rowraw_v, rowidx_v, rowbuf_v, outbuf_v, sem):
    wid = lax.axis_index("s") * NC + lax.axis_index("c")
    base = wid * RPB
    for b in range(B):
        pltpu.sync_copy(idx_hbm.at[b], colidx_v)
        pltpu.sync_copy(idx_hbm.at[b, pl.ds(base, RPB)], rowraw_v)
        for m in range(RPB // 16):
            rowidx_v[pl.ds(m * 16, 16)] = (
                rowraw_v[pl.ds(m * 16, 16)] + b * N)

        def chunk_body(cc, carry):
            cp = pltpu.make_async_copy(
                adj_hbm.at[rowidx_v.at[pl.ds(cc * CHUNK, CHUNK)]],
                rowbuf_v, sem)
            cp.start()
            cp.wait()

            def j_body(j, carry2):
                cidx = colidx_v[pl.ds(j * 16, 16)]
                for rloc in range(CHUNK):
                    vals = plsc.load_gather(rowbuf_v.at[rloc], [cidx])
                    outbuf_v[rloc, pl.ds(j * 16, 16)] = vals
                return carry2

            lax.fori_loop(0, K // 16, j_body, 0, unroll=2)
            pltpu.sync_copy(
                outbuf_v,
                out_hbm.at[pl.ds(b * K + base + cc * CHUNK, CHUNK)])
            return carry

        lax.fori_loop(0, RPB // CHUNK, chunk_body, 0)


@functools.cache
def _adj_gather_sc():
    return pl.kernel(
        _adj_sc_body,
        out_type=jax.ShapeDtypeStruct((B * K, K), jnp.float32),
        mesh=plsc.VectorSubcoreMesh(core_axis_name="c", subcore_axis_name="s",
                                    num_cores=NC, num_subcores=NS),
        compiler_params=pltpu.CompilerParams(use_tc_tiling_on_sc=False,
                                             needs_layout_passes=False),
        scratch_types=[
            pltpu.VMEM((K,), jnp.int32),
            pltpu.VMEM((RPB,), jnp.int32),
            pltpu.VMEM((RPB,), jnp.int32),
            pltpu.VMEM((CHUNK, N), jnp.float32),
            pltpu.VMEM((CHUNK, K), jnp.float32),
            pltpu.SemaphoreType.DMA,
        ],
    )


def kernel(node_embed, adj_matrix, weight_p):
    pooled, idx3 = _topk_pool_tc(node_embed, weight_p)
    idx = idx3.reshape(B, K)
    adjflat = adj_matrix.reshape(B * N, N)
    outflat = _adj_gather_sc()(adjflat, idx)
    return pooled, outflat.reshape(B, K, K)


# speedup vs baseline: 1.0198x; 1.0022x over previous
import functools

import jax
import jax.numpy as jnp
from jax import lax
from jax.experimental import pallas as pl
from jax.experimental.pallas import tpu as pltpu
from jax.experimental.pallas import tpu_sc as plsc

B, N, F = 2, 4096, 128
K = 2048
RT = 4
RTS = K // RT
JT = 8
JTS = N // JT

NC = 2
NS = 16
NW = NC * NS
ROWS_PER_W = (B * K) // NW
RPB = K // NW
CHUNK = 8


def _topk_tc_body(x_ref, w_ref, pooled_ref, idx_ref, srow_s, scol_s, rank_s):
    rt = pl.program_id(1)
    x = x_ref[0]

    @pl.when(rt == 0)
    def _():
        w = w_ref[...]
        scol = jnp.dot(x, w)
        srow = lax.dot_general(w, x, (((0,), (1,)), ((), ())))
        scol_s[...] = scol
        srow_s[...] = srow
        ii = lax.broadcasted_iota(jnp.int32, (N, 1), 0)
        ones = jnp.ones((JTS, 1), jnp.float32)

        def rank_step(jt, acc):
            sj = srow_s[:, pl.ds(jt * JTS, JTS)]
            jj = jt * JTS + lax.broadcasted_iota(jnp.int32, (1, JTS), 1)
            gt = sj > scol
            tie = (sj == scol) & (jj < ii)
            contrib = jnp.where(gt | tie, 1.0, 0.0)
            return acc + jnp.dot(contrib, ones,
                                 precision=lax.Precision.HIGHEST)

        rank_s[...] = lax.fori_loop(
            0, JT, rank_step, jnp.zeros((N, 1), jnp.float32))

    rank = rank_s[...]
    rr = (rt * RTS + lax.broadcasted_iota(jnp.int32, (1, RTS), 1)
          ).astype(jnp.float32)
    sel = jnp.where(rank == rr, 1.0, 0.0)
    iota_row = lax.broadcasted_iota(jnp.int32, (1, N), 1).astype(jnp.float32)
    idx_f = jnp.dot(iota_row, sel, precision=lax.Precision.HIGHEST)
    idx_ref[0] = idx_f.astype(jnp.int32)
    vals = lax.dot_general(sel, scol_s[...], (((0,), (0,)), ((), ())),
                           precision=lax.Precision.HIGHEST)
    pooled = lax.dot_general(sel, x, (((0,), (0,)), ((), ())),
                             precision=lax.Precision.HIGHEST)
    pooled_ref[0] = pooled * jnp.tanh(vals)


def _topk_pool_tc(node_embed, weight_p, interpret=False):
    return pl.pallas_call(
        _topk_tc_body,
        grid=(B, RT),
        in_specs=[
            pl.BlockSpec((1, N, F), lambda b, rt: (b, 0, 0)),
            pl.BlockSpec((F, 1), lambda b, rt: (0, 0)),
        ],
        out_specs=[
            pl.BlockSpec((1, RTS, F), lambda b, rt: (b, rt, 0)),
            pl.BlockSpec((1, 1, RTS), lambda b, rt: (b, 0, rt)),
        ],
        out_shape=[
            jax.ShapeDtypeStruct((B, K, F), jnp.float32),
            jax.ShapeDtypeStruct((B, 1, K), jnp.int32),
        ],
        scratch_shapes=[
            pltpu.VMEM((1, N), jnp.float32),
            pltpu.VMEM((N, 1), jnp.float32),
            pltpu.VMEM((N, 1), jnp.float32),
        ],
        compiler_params=None if interpret else pltpu.CompilerParams(
            vmem_limit_bytes=60 * 1024 * 1024),
        interpret=interpret,
    )(node_embed, weight_p)


def _adj_sc_body(adj_hbm, idx_hbm, out_hbm,
                 colidx_v, rowbuf_v, outbuf_v, sem):
    wid = lax.axis_index("s") * NC + lax.axis_index("c")
    base = wid * RPB
    for b in range(B):
        pltpu.sync_copy(idx_hbm.at[b], colidx_v)

        def chunk_body(cc, carry):
            cp = pltpu.make_async_copy(
                adj_hbm.at[b].at[colidx_v.at[pl.ds(base + cc * CHUNK, CHUNK)]],
                rowbuf_v, sem)
            cp.start()
            cp.wait()

            def j_body(j, carry2):
                cidx = colidx_v[pl.ds(j * 16, 16)]
                for rloc in range(CHUNK):
                    vals = plsc.load_gather(rowbuf_v.at[rloc], [cidx])
                    outbuf_v[rloc, pl.ds(j * 16, 16)] = vals
                return carry2

            lax.fori_loop(0, K // 16, j_body, 0, unroll=2)
            pltpu.sync_copy(
                outbuf_v,
                out_hbm.at[b].at[pl.ds(base + cc * CHUNK, CHUNK)])
            return carry

        lax.fori_loop(0, RPB // CHUNK, chunk_body, 0)


@functools.cache
def _adj_gather_sc():
    return pl.kernel(
        _adj_sc_body,
        out_type=jax.ShapeDtypeStruct((B, K, K), jnp.float32),
        mesh=plsc.VectorSubcoreMesh(core_axis_name="c", subcore_axis_name="s",
                                    num_cores=NC, num_subcores=NS),
        compiler_params=pltpu.CompilerParams(use_tc_tiling_on_sc=False,
                                             needs_layout_passes=False),
        scratch_types=[
            pltpu.VMEM((K,), jnp.int32),
            pltpu.VMEM((CHUNK, N), jnp.float32),
            pltpu.VMEM((CHUNK, K), jnp.float32),
            pltpu.SemaphoreType.DMA,
        ],
    )


def kernel(node_embed, adj_matrix, weight_p):
    pooled, idx3 = _topk_pool_tc(node_embed, weight_p)
    idx = idx3.reshape(B, K)
    adj_pooled = _adj_gather_sc()(adj_matrix, idx)
    return pooled, adj_pooled


# speedup vs baseline: 1.3914x; 1.3643x over previous
import functools

import jax
import jax.numpy as jnp
from jax import lax
from jax.experimental import pallas as pl
from jax.experimental.pallas import tpu as pltpu
from jax.experimental.pallas import tpu_sc as plsc

B, N, F = 2, 4096, 128
K = 2048
JT = 8
JTS = N // JT

NC = 2
NS = 16
NW = NC * NS
RPB = K // NW
CHUNK = 8


def _rank_tc_body(x_ref, w_ref, rank_ref, tanh_ref, srow_s):
    x = x_ref[0]
    w = w_ref[...]
    scol = jnp.dot(x, w)
    srow = lax.dot_general(w, x, (((0,), (1,)), ((), ())))
    srow_s[...] = srow
    ii = lax.broadcasted_iota(jnp.int32, (N, 1), 0)
    ones = jnp.ones((JTS, 1), jnp.float32)

    def rank_step(jt, acc):
        sj = srow_s[:, pl.ds(jt * JTS, JTS)]
        jj = jt * JTS + lax.broadcasted_iota(jnp.int32, (1, JTS), 1)
        gt = sj > scol
        tie = (sj == scol) & (jj < ii)
        contrib = jnp.where(gt | tie, 1.0, 0.0)
        return acc + jnp.dot(contrib, ones, precision=lax.Precision.HIGHEST)

    rank = lax.fori_loop(0, JT, rank_step, jnp.zeros((N, 1), jnp.float32))
    rank_ref[0] = rank.astype(jnp.int32)
    tanh_ref[0] = jnp.tanh(scol)


def _rank_tc(node_embed, weight_p, interpret=False):
    return pl.pallas_call(
        _rank_tc_body,
        grid=(B,),
        in_specs=[
            pl.BlockSpec((1, N, F), lambda b: (b, 0, 0)),
            pl.BlockSpec((F, 1), lambda b: (0, 0)),
        ],
        out_specs=[
            pl.BlockSpec((1, N, 1), lambda b: (b, 0, 0)),
            pl.BlockSpec((1, N, 1), lambda b: (b, 0, 0)),
        ],
        out_shape=[
            jax.ShapeDtypeStruct((B, N, 1), jnp.int32),
            jax.ShapeDtypeStruct((B, N, 1), jnp.float32),
        ],
        scratch_shapes=[pltpu.VMEM((1, N), jnp.float32)],
        compiler_params=None if interpret else pltpu.CompilerParams(
            vmem_limit_bytes=60 * 1024 * 1024),
        interpret=interpret,
    )(node_embed, weight_p)


def _pool_sc_body(adj_hbm, rank_hbm, tanh_hbm, emb_hbm, out_hbm, pool_hbm,
                  rankbuf_v, tbuf_v, colidx_v, trow_v,
                  rowbuf_v, outbuf_v, embrows_v, sem):
    wid = lax.axis_index("s") * NC + lax.axis_index("c")
    base = wid * RPB
    i16 = lax.broadcasted_iota(jnp.int32, (16,), 0)
    for b in range(B):
        pltpu.sync_copy(rank_hbm.at[b], rankbuf_v)
        pltpu.sync_copy(tanh_hbm.at[b], tbuf_v)
        for m in range(N // 16):
            rk = rankbuf_v[pl.ds(m * 16, 16)]
            plsc.store_scatter(colidx_v, [rk], i16 + (m * 16),
                               mask=rk < K)

        cp = pltpu.make_async_copy(
            emb_hbm.at[b].at[colidx_v.at[pl.ds(base, RPB)]],
            embrows_v, sem)
        cp.start()
        cp.wait()
        for m in range(RPB // 16):
            idx16 = colidx_v[pl.ds(base + m * 16, 16)]
            trow_v[pl.ds(m * 16, 16)] = plsc.load_gather(tbuf_v, [idx16])

        def row_body(r, carry):
            t16 = plsc.load_gather(trow_v, [jnp.full((16,), r, jnp.int32)])
            for c in range(F // 16):
                embrows_v[r, pl.ds(c * 16, 16)] = (
                    embrows_v[r, pl.ds(c * 16, 16)] * t16)
            return carry

        lax.fori_loop(0, RPB, row_body, 0, unroll=2)
        pltpu.sync_copy(embrows_v, pool_hbm.at[b].at[pl.ds(base, RPB)])

        def chunk_body(cc, carry):
            cp2 = pltpu.make_async_copy(
                adj_hbm.at[b].at[colidx_v.at[pl.ds(base + cc * CHUNK, CHUNK)]],
                rowbuf_v, sem)
            cp2.start()
            cp2.wait()

            def j_body(j, carry2):
                cidx = colidx_v[pl.ds(j * 16, 16)]
                for rloc in range(CHUNK):
                    vals = plsc.load_gather(rowbuf_v.at[rloc], [cidx])
                    outbuf_v[rloc, pl.ds(j * 16, 16)] = vals
                return carry2

            lax.fori_loop(0, K // 16, j_body, 0, unroll=2)
            pltpu.sync_copy(
                outbuf_v,
                out_hbm.at[b].at[pl.ds(base + cc * CHUNK, CHUNK)])
            return carry

        lax.fori_loop(0, RPB // CHUNK, chunk_body, 0)


@functools.cache
def _pool_sc():
    return pl.kernel(
        _pool_sc_body,
        out_type=[
            jax.ShapeDtypeStruct((B, K, K), jnp.float32),
            jax.ShapeDtypeStruct((B, K, F), jnp.float32),
        ],
        mesh=plsc.VectorSubcoreMesh(core_axis_name="c", subcore_axis_name="s",
                                    num_cores=NC, num_subcores=NS),
        compiler_params=pltpu.CompilerParams(use_tc_tiling_on_sc=False,
                                             needs_layout_passes=False),
        scratch_types=[
            pltpu.VMEM((N,), jnp.int32),
            pltpu.VMEM((N,), jnp.float32),
            pltpu.VMEM((K,), jnp.int32),
            pltpu.VMEM((RPB,), jnp.float32),
            pltpu.VMEM((CHUNK, N), jnp.float32),
            pltpu.VMEM((CHUNK, K), jnp.float32),
            pltpu.VMEM((RPB, F), jnp.float32),
            pltpu.SemaphoreType.DMA,
        ],
    )


def kernel(node_embed, adj_matrix, weight_p):
    rank3, tanh3 = _rank_tc(node_embed, weight_p)
    rank = rank3.reshape(B, N)
    tanhs = tanh3.reshape(B, N)
    adj_pooled, pooled = _pool_sc()(adj_matrix, rank, tanhs, node_embed)
    return pooled, adj_pooled
